# baseline (speedup 1.0000x reference)
import functools

import jax
import jax.numpy as jnp
from jax import lax
from jax.experimental import pallas as pl
from jax.experimental.pallas import tpu as pltpu
from jax.experimental.pallas import tpu_sc as plsc

B, N, D = 32, 4096, 128
NC, NS, L = 2, 16, 16
NW = NC * NS
ROWS_PER_CHUNK = 128
NUM_CHUNKS = N // ROWS_PER_CHUNK
ROW_TILE = 512
NUM_ROW_TILES = N // ROW_TILE


def _keys_body(x_ref, keys_ref):
    vt = x_ref[0].T
    acc = vt[0:8, :]
    for k in range(1, D // 8):
        acc = acc + vt[8 * k:8 * (k + 1), :]
    t = acc[0:4, :] + acc[4:8, :]
    t = t[0:2, :] + t[2:4, :]
    keys_ref[0, 0, :] = t[0, :] + t[1, :]


def _rank_body(keys_ref, rank_ref):
    k_all = keys_ref[0, 0, :]
    one = jnp.ones((ROW_TILE, ROW_TILE), jnp.int32)
    zero = jnp.zeros((ROW_TILE, ROW_TILE), jnp.int32)
    for ti in range(NUM_ROW_TILES):
        ki = lax.slice(k_all, (ti * ROW_TILE,), ((ti + 1) * ROW_TILE,))[:, None]
        acc = jnp.zeros((ROW_TILE, ROW_TILE), jnp.int32)
        for tj in range(NUM_ROW_TILES):
            kj = lax.slice(k_all, (tj * ROW_TILE,), ((tj + 1) * ROW_TILE,))[None, :]
            if tj < ti:
                c = kj <= ki
            elif tj > ti:
                c = kj < ki
            else:
                j_idx = lax.broadcasted_iota(jnp.int32, (ROW_TILE, ROW_TILE), 1)
                i_idx = lax.broadcasted_iota(jnp.int32, (ROW_TILE, ROW_TILE), 0)
                c = (kj < ki) | ((kj == ki) & (j_idx < i_idx))
            acc = acc + jnp.where(c, one, zero)
        rank_ref[0, 0, pl.ds(ti * ROW_TILE, ROW_TILE)] = jnp.sum(acc, axis=1)


def _sc_body(x_hbm, rank_hbm, out_hbm, rank_v, idx_v, rows_v, sem):
    w = lax.axis_index("s") * NC + lax.axis_index("c")
    base = w * N

    pltpu.sync_copy(rank_hbm.at[w], rank_v)

    def inv_body(c, carry):
        r = rank_v[pl.ds(c * L, L)]
        val = base + c * L + lax.iota(jnp.int32, L)
        plsc.store_scatter(idx_v, [r], val)
        return carry

    lax.fori_loop(0, N // L, inv_body, 0)

    def g_body(j, carry):
        pltpu.async_copy(
            x_hbm.at[idx_v.at[pl.ds(j * ROWS_PER_CHUNK, ROWS_PER_CHUNK)]],
            rows_v, sem).wait()
        pltpu.sync_copy(rows_v, out_hbm.at[pl.ds(base + j * ROWS_PER_CHUNK,
                                                 ROWS_PER_CHUNK)])
        return carry

    lax.fori_loop(0, NUM_CHUNKS, g_body, 0)


@jax.jit
def kernel(x):
    keys = pl.pallas_call(
        _keys_body,
        grid=(B,),
        in_specs=[pl.BlockSpec((1, N, D), lambda b: (b, 0, 0))],
        out_specs=pl.BlockSpec((1, 1, N), lambda b: (b, 0, 0)),
        out_shape=jax.ShapeDtypeStruct((B, 1, N), jnp.float32),
    )(x)

    rank = pl.pallas_call(
        _rank_body,
        grid=(B,),
        in_specs=[pl.BlockSpec((1, 1, N), lambda b: (b, 0, 0))],
        out_specs=pl.BlockSpec((1, 1, N), lambda b: (b, 0, 0)),
        out_shape=jax.ShapeDtypeStruct((B, 1, N), jnp.int32),
    )(keys).reshape(B, N)

    x_flat = x.reshape(B * N, D)
    mesh = plsc.VectorSubcoreMesh(core_axis_name="c", subcore_axis_name="s")
    out_flat = pl.kernel(
        _sc_body,
        out_type=jax.ShapeDtypeStruct((B * N, D), jnp.float32),
        mesh=mesh,
        compiler_params=pltpu.CompilerParams(needs_layout_passes=False),
        scratch_types=[
            pltpu.VMEM((N,), jnp.int32),
            pltpu.VMEM((N,), jnp.int32),
            pltpu.VMEM((ROWS_PER_CHUNK, D), jnp.float32),
            pltpu.SemaphoreType.DMA,
        ],
    )(x_flat, rank)
    return out_flat.reshape(B, N, D)


# speedup vs baseline: 3.0608x; 3.0608x over previous
import jax
import jax.numpy as jnp
import numpy as np
from jax import lax
from jax.experimental import pallas as pl
from jax.experimental.pallas import tpu as pltpu
from jax.experimental.pallas import tpu_sc as plsc

B, N, D = 32, 4096, 128
NC, NS, L = 2, 16, 16
ROWS_PER_CHUNK = 128
NUM_CHUNKS = N // ROWS_PER_CHUNK
NCHUNK16 = N // L
HIST = 2048
SIGN = np.int32(-2147483648)


def _keys_body(x_ref, keys_ref):
    vt = x_ref[0].T
    acc = vt[0:8, :]
    for k in range(1, D // 8):
        acc = acc + vt[8 * k:8 * (k + 1), :]
    t = acc[0:4, :] + acc[4:8, :]
    t = t[0:2, :] + t[2:4, :]
    keys_ref[0, 0, :] = t[0, :] + t[1, :]


def _radix_pass(shift, src_su, src_id, dst_su, dst_id, hist_v):
    @pl.loop(0, HIST // L)
    def _zero(h):
        hist_v[pl.ds(h * L, L)] = jnp.zeros((L,), jnp.int32)

    @pl.loop(0, NCHUNK16)
    def _hist(c):
        su = src_su[pl.ds(c * L, L)]
        d = lax.bitwise_and(lax.shift_right_logical(su, shift), 2047)
        h0 = plsc.load_gather(hist_v, [d])
        cnt, last = plsc.scan_count(d)
        plsc.store_scatter(hist_v, [d], h0 + cnt, mask=last)

    @pl.loop(0, HIST // L, init_carry=np.int32(0))
    def _prefix(h, carry):
        v = hist_v[pl.ds(h * L, L)]
        cs = plsc.cumsum(v)
        hist_v[pl.ds(h * L, L)] = cs - v + carry
        return carry + jnp.sum(v)

    @pl.loop(0, NCHUNK16)
    def _place(c):
        su = src_su[pl.ds(c * L, L)]
        iv = src_id[pl.ds(c * L, L)]
        d = lax.bitwise_and(lax.shift_right_logical(su, shift), 2047)
        off = plsc.load_gather(hist_v, [d])
        cnt, last = plsc.scan_count(d)
        pos = off + cnt - 1
        plsc.store_scatter(dst_su, [pos], su)
        plsc.store_scatter(dst_id, [pos], iv)
        plsc.store_scatter(hist_v, [d], off + cnt, mask=last)


def _sc_body(x_hbm, keys_hbm, out_hbm,
             kf_v, su_a, id_a, su_b, id_b, hist_v, rows0, rows1, sem0, sem1):
    w = lax.axis_index("s") * NC + lax.axis_index("c")
    base = w * N

    pltpu.sync_copy(keys_hbm.at[w], kf_v)

    @pl.loop(0, NCHUNK16)
    def _init(c):
        s = plsc.bitcast(kf_v[pl.ds(c * L, L)], jnp.int32)
        m = lax.bitwise_or(lax.shift_right_arithmetic(s, 31), SIGN)
        su_a[pl.ds(c * L, L)] = lax.bitwise_xor(s, m)
        id_a[pl.ds(c * L, L)] = base + c * L + lax.iota(jnp.int32, L)

    _radix_pass(0, su_a, id_a, su_b, id_b, hist_v)
    _radix_pass(11, su_b, id_b, su_a, id_a, hist_v)
    _radix_pass(22, su_a, id_a, su_b, id_b, hist_v)

    def _gather(c, rows, sem):
        return pltpu.async_copy(
            x_hbm.at[id_b.at[pl.ds(c * ROWS_PER_CHUNK, ROWS_PER_CHUNK)]],
            rows, sem)

    def _drain(rows, sem):
        pltpu.make_async_copy(x_hbm.at[pl.ds(0, ROWS_PER_CHUNK)], rows,
                              sem).wait()

    _gather(0, rows0, sem0)

    @pl.loop(0, NUM_CHUNKS, step=2)
    def _g(c):
        _gather(c + 1, rows1, sem1)
        _drain(rows0, sem0)
        pltpu.sync_copy(rows0, out_hbm.at[pl.ds(base + c * ROWS_PER_CHUNK,
                                                ROWS_PER_CHUNK)])

        @pl.when(c + 2 < NUM_CHUNKS)
        def _():
            _gather(c + 2, rows0, sem0)

        _drain(rows1, sem1)
        pltpu.sync_copy(rows1, out_hbm.at[pl.ds(base + (c + 1) * ROWS_PER_CHUNK,
                                                ROWS_PER_CHUNK)])


@jax.jit
def kernel(x):
    keys = pl.pallas_call(
        _keys_body,
        grid=(B,),
        in_specs=[pl.BlockSpec((1, N, D), lambda b: (b, 0, 0))],
        out_specs=pl.BlockSpec((1, 1, N), lambda b: (b, 0, 0)),
        out_shape=jax.ShapeDtypeStruct((B, 1, N), jnp.float32),
    )(x).reshape(B, N)

    x_flat = x.reshape(B * N, D)
    mesh = plsc.VectorSubcoreMesh(core_axis_name="c", subcore_axis_name="s")
    out_flat = pl.kernel(
        _sc_body,
        out_type=jax.ShapeDtypeStruct((B * N, D), jnp.float32),
        mesh=mesh,
        compiler_params=pltpu.CompilerParams(needs_layout_passes=False),
        scratch_types=[
            pltpu.VMEM((N,), jnp.float32),
            pltpu.VMEM((N,), jnp.int32),
            pltpu.VMEM((N,), jnp.int32),
            pltpu.VMEM((N,), jnp.int32),
            pltpu.VMEM((N,), jnp.int32),
            pltpu.VMEM((HIST,), jnp.int32),
            pltpu.VMEM((ROWS_PER_CHUNK, D), jnp.float32),
            pltpu.VMEM((ROWS_PER_CHUNK, D), jnp.float32),
            pltpu.SemaphoreType.DMA,
            pltpu.SemaphoreType.DMA,
        ],
    )(x_flat, keys)
    return out_flat.reshape(B, N, D)


# speedup vs baseline: 3.1551x; 1.0308x over previous
import jax
import jax.numpy as jnp
import numpy as np
from jax import lax
from jax.experimental import pallas as pl
from jax.experimental.pallas import tpu as pltpu
from jax.experimental.pallas import tpu_sc as plsc

B, N, D = 32, 4096, 128
NC, NS, L = 2, 16, 16
ROWS_PER_CHUNK = 128
NUM_CHUNKS = N // ROWS_PER_CHUNK
NCHUNK16 = N // L
HIST = 2048
SIGN = np.int32(-2147483648)


def _keys_body(x_ref, keys_ref):
    vt = x_ref[0].T
    acc = vt[0:8, :]
    for k in range(1, D // 8):
        acc = acc + vt[8 * k:8 * (k + 1), :]
    t = acc[0:4, :] + acc[4:8, :]
    t = t[0:2, :] + t[2:4, :]
    keys_ref[0, 0, :] = t[0, :] + t[1, :]


def _sortable(kf):
    s = plsc.bitcast(kf, jnp.int32)
    m = lax.bitwise_or(lax.shift_right_arithmetic(s, 31), SIGN)
    return lax.bitwise_xor(s, m)


def _radix_pass(shift, load_su, load_id, dst_su, dst_id, hist_v):
    @pl.loop(0, HIST // L, unroll=8)
    def _zero(h):
        hist_v[pl.ds(h * L, L)] = jnp.zeros((L,), jnp.int32)

    @pl.loop(0, NCHUNK16, unroll=4)
    def _hist(c):
        d = lax.bitwise_and(lax.shift_right_logical(load_su(c), shift), 2047)
        h0 = plsc.load_gather(hist_v, [d])
        cnt, last = plsc.scan_count(d)
        plsc.store_scatter(hist_v, [d], h0 + cnt, mask=last)

    @pl.loop(0, HIST // L, init_carry=np.int32(0), unroll=2)
    def _prefix(h, carry):
        v = hist_v[pl.ds(h * L, L)]
        cs = plsc.cumsum(v)
        hist_v[pl.ds(h * L, L)] = cs - v + carry
        return carry + jnp.sum(v)

    @pl.loop(0, NCHUNK16, unroll=4)
    def _place(c):
        su = load_su(c)
        iv = load_id(c)
        d = lax.bitwise_and(lax.shift_right_logical(su, shift), 2047)
        off = plsc.load_gather(hist_v, [d])
        cnt, last = plsc.scan_count(d)
        pos = off + cnt - 1
        plsc.store_scatter(dst_su, [pos], su)
        plsc.store_scatter(dst_id, [pos], iv)
        plsc.store_scatter(hist_v, [d], off + cnt, mask=last)


def _sc_body(x_hbm, keys_hbm, out_hbm,
             kf_v, su_a, id_a, su_b, id_b, hist_v,
             rows0, rows1, rows2, rows3,
             g0, g1, g2, g3, w0, w1, w2, w3):
    w = lax.axis_index("s") * NC + lax.axis_index("c")
    base = w * N

    pltpu.sync_copy(keys_hbm.at[w], kf_v)

    _radix_pass(
        0,
        lambda c: _sortable(kf_v[pl.ds(c * L, L)]),
        lambda c: base + c * L + lax.iota(jnp.int32, L),
        su_b, id_b, hist_v)
    _radix_pass(11,
                lambda c: su_b[pl.ds(c * L, L)],
                lambda c: id_b[pl.ds(c * L, L)],
                su_a, id_a, hist_v)
    _radix_pass(22,
                lambda c: su_a[pl.ds(c * L, L)],
                lambda c: id_a[pl.ds(c * L, L)],
                su_b, id_b, hist_v)

    rows = [rows0, rows1, rows2, rows3]
    gsem = [g0, g1, g2, g3]
    wsem = [w0, w1, w2, w3]

    def _gather_start(q, j):
        pltpu.async_copy(
            x_hbm.at[id_b.at[pl.ds(q * ROWS_PER_CHUNK, ROWS_PER_CHUNK)]],
            rows[j], gsem[j])

    def _gather_drain(j):
        pltpu.make_async_copy(x_hbm.at[pl.ds(0, ROWS_PER_CHUNK)], rows[j],
                              gsem[j]).wait()

    def _write_start(q, j):
        pltpu.async_copy(
            rows[j], out_hbm.at[pl.ds(base + q * ROWS_PER_CHUNK,
                                      ROWS_PER_CHUNK)], wsem[j])

    def _write_drain(j):
        pltpu.make_async_copy(x_hbm.at[pl.ds(0, ROWS_PER_CHUNK)], rows[j],
                              wsem[j]).wait()

    _gather_start(0, 0)
    _gather_start(1, 1)

    @pl.loop(0, NUM_CHUNKS, step=4)
    def _g(c):
        for jj in range(4):
            q = c + jj
            j = jj
            j2 = (jj + 2) % 4
            _gather_drain(j)
            _write_start(q, j)

            @pl.when(q + 2 < NUM_CHUNKS)
            def _():
                @pl.when(q >= 2)
                def _():
                    _write_drain(j2)
                _gather_start(q + 2, j2)

    _write_drain(0)
    _write_drain(1)
    _write_drain(2)
    _write_drain(3)


@jax.jit
def kernel(x):
    keys = pl.pallas_call(
        _keys_body,
        grid=(B,),
        in_specs=[pl.BlockSpec((1, N, D), lambda b: (b, 0, 0))],
        out_specs=pl.BlockSpec((1, 1, N), lambda b: (b, 0, 0)),
        out_shape=jax.ShapeDtypeStruct((B, 1, N), jnp.float32),
    )(x).reshape(B, N)

    x_flat = x.reshape(B * N, D)
    mesh = plsc.VectorSubcoreMesh(core_axis_name="c", subcore_axis_name="s")
    out_flat = pl.kernel(
        _sc_body,
        out_type=jax.ShapeDtypeStruct((B * N, D), jnp.float32),
        mesh=mesh,
        compiler_params=pltpu.CompilerParams(needs_layout_passes=False),
        scratch_types=[
            pltpu.VMEM((N,), jnp.float32),
            pltpu.VMEM((N,), jnp.int32),
            pltpu.VMEM((N,), jnp.int32),
            pltpu.VMEM((N,), jnp.int32),
            pltpu.VMEM((N,), jnp.int32),
            pltpu.VMEM((HIST,), jnp.int32),
            pltpu.VMEM((ROWS_PER_CHUNK, D), jnp.float32),
            pltpu.VMEM((ROWS_PER_CHUNK, D), jnp.float32),
            pltpu.VMEM((ROWS_PER_CHUNK, D), jnp.float32),
            pltpu.VMEM((ROWS_PER_CHUNK, D), jnp.float32),
            pltpu.SemaphoreType.DMA,
            pltpu.SemaphoreType.DMA,
            pltpu.SemaphoreType.DMA,
            pltpu.SemaphoreType.DMA,
            pltpu.SemaphoreType.DMA,
            pltpu.SemaphoreType.DMA,
            pltpu.SemaphoreType.DMA,
            pltpu.SemaphoreType.DMA,
        ],
    )(x_flat, keys)
    return out_flat.reshape(B, N, D)


# speedup vs baseline: 3.1865x; 1.0100x over previous
import jax
import jax.numpy as jnp
import numpy as np
from jax import lax
from jax.experimental import pallas as pl
from jax.experimental.pallas import tpu as pltpu
from jax.experimental.pallas import tpu_sc as plsc

B, N, D = 32, 4096, 128
NC, NS, L = 2, 16, 16
ROWS_PER_CHUNK = 64
NUM_CHUNKS = N // ROWS_PER_CHUNK
NCHUNK16 = N // L
HIST = 2048
SIGN = np.int32(-2147483648)


def _keys_body(x_ref, keys_ref):
    vt = x_ref[0].T
    acc = vt[0:8, :]
    for k in range(1, D // 8):
        acc = acc + vt[8 * k:8 * (k + 1), :]
    t = acc[0:4, :] + acc[4:8, :]
    t = t[0:2, :] + t[2:4, :]
    keys_ref[0, 0, :] = t[0, :] + t[1, :]


def _sortable(kf):
    s = plsc.bitcast(kf, jnp.int32)
    m = lax.bitwise_or(lax.shift_right_arithmetic(s, 31), SIGN)
    return lax.bitwise_xor(s, m)


def _radix_pass(shift, load_su, load_id, dst_su, dst_id, hist_v):
    @pl.loop(0, HIST // L, unroll=8)
    def _zero(h):
        hist_v[pl.ds(h * L, L)] = jnp.zeros((L,), jnp.int32)

    @pl.loop(0, NCHUNK16, unroll=4)
    def _hist(c):
        d = lax.bitwise_and(lax.shift_right_logical(load_su(c), shift), 2047)
        h0 = plsc.load_gather(hist_v, [d])
        cnt, last = plsc.scan_count(d)
        plsc.store_scatter(hist_v, [d], h0 + cnt, mask=last)

    @pl.loop(0, HIST // L, init_carry=np.int32(0), unroll=2)
    def _prefix(h, carry):
        v = hist_v[pl.ds(h * L, L)]
        cs = plsc.cumsum(v)
        hist_v[pl.ds(h * L, L)] = cs - v + carry
        return carry + jnp.sum(v)

    @pl.loop(0, NCHUNK16, unroll=4)
    def _place(c):
        su = load_su(c)
        iv = load_id(c)
        d = lax.bitwise_and(lax.shift_right_logical(su, shift), 2047)
        off = plsc.load_gather(hist_v, [d])
        cnt, last = plsc.scan_count(d)
        pos = off + cnt - 1
        plsc.store_scatter(dst_su, [pos], su)
        plsc.store_scatter(dst_id, [pos], iv)
        plsc.store_scatter(hist_v, [d], off + cnt, mask=last)


NSLOT = 8
PREFETCH = 4


def _sc_body(x_hbm, keys_hbm, out_hbm,
             kf_v, su_a, id_a, su_b, id_b, hist_v,
             rows0, rows1, rows2, rows3, rows4, rows5, rows6, rows7,
             g0, g1, g2, g3, g4, g5, g6, g7,
             w0, w1, w2, w3, w4, w5, w6, w7):
    w = lax.axis_index("s") * NC + lax.axis_index("c")
    base = w * N

    pltpu.sync_copy(keys_hbm.at[w], kf_v)

    _radix_pass(
        0,
        lambda c: _sortable(kf_v[pl.ds(c * L, L)]),
        lambda c: base + c * L + lax.iota(jnp.int32, L),
        su_b, id_b, hist_v)
    _radix_pass(11,
                lambda c: su_b[pl.ds(c * L, L)],
                lambda c: id_b[pl.ds(c * L, L)],
                su_a, id_a, hist_v)
    _radix_pass(22,
                lambda c: su_a[pl.ds(c * L, L)],
                lambda c: id_a[pl.ds(c * L, L)],
                su_b, id_b, hist_v)

    rows = [rows0, rows1, rows2, rows3, rows4, rows5, rows6, rows7]
    gsem = [g0, g1, g2, g3, g4, g5, g6, g7]
    wsem = [w0, w1, w2, w3, w4, w5, w6, w7]

    def _gather_start(q, j):
        pltpu.async_copy(
            x_hbm.at[id_b.at[pl.ds(q * ROWS_PER_CHUNK, ROWS_PER_CHUNK)]],
            rows[j], gsem[j])

    def _gather_drain(j):
        pltpu.make_async_copy(x_hbm.at[pl.ds(0, ROWS_PER_CHUNK)], rows[j],
                              gsem[j]).wait()

    def _write_start(q, j):
        pltpu.async_copy(
            rows[j], out_hbm.at[pl.ds(base + q * ROWS_PER_CHUNK,
                                      ROWS_PER_CHUNK)], wsem[j])

    def _write_drain(j):
        pltpu.make_async_copy(x_hbm.at[pl.ds(0, ROWS_PER_CHUNK)], rows[j],
                              wsem[j]).wait()

    for k in range(PREFETCH):
        _gather_start(k, k)

    @pl.loop(0, NUM_CHUNKS, step=NSLOT)
    def _g(c):
        for jj in range(NSLOT):
            q = c + jj
            j = jj
            j2 = (jj + PREFETCH) % NSLOT
            _gather_drain(j)
            _write_start(q, j)

            @pl.when(q + PREFETCH < NUM_CHUNKS)
            def _():
                @pl.when(q >= NSLOT - PREFETCH)
                def _():
                    _write_drain(j2)
                _gather_start(q + PREFETCH, j2)

    for k in range(NSLOT):
        _write_drain((NUM_CHUNKS - NSLOT + k) % NSLOT)


@jax.jit
def kernel(x):
    keys = pl.pallas_call(
        _keys_body,
        grid=(B,),
        in_specs=[pl.BlockSpec((1, N, D), lambda b: (b, 0, 0))],
        out_specs=pl.BlockSpec((1, 1, N), lambda b: (b, 0, 0)),
        out_shape=jax.ShapeDtypeStruct((B, 1, N), jnp.float32),
    )(x).reshape(B, N)

    x_flat = x.reshape(B * N, D)
    mesh = plsc.VectorSubcoreMesh(core_axis_name="c", subcore_axis_name="s")
    out_flat = pl.kernel(
        _sc_body,
        out_type=jax.ShapeDtypeStruct((B * N, D), jnp.float32),
        mesh=mesh,
        compiler_params=pltpu.CompilerParams(needs_layout_passes=False),
        scratch_types=[
            pltpu.VMEM((N,), jnp.float32),
            pltpu.VMEM((N,), jnp.int32),
            pltpu.VMEM((N,), jnp.int32),
            pltpu.VMEM((N,), jnp.int32),
            pltpu.VMEM((N,), jnp.int32),
            pltpu.VMEM((HIST,), jnp.int32),
        ] + [pltpu.VMEM((ROWS_PER_CHUNK, D), jnp.float32)] * NSLOT
          + [pltpu.SemaphoreType.DMA] * (2 * NSLOT),
    )(x_flat, keys)
    return out_flat.reshape(B, N, D)
